# baseline (speedup 1.0000x reference)
import functools

import jax
import jax.numpy as jnp
from jax import lax
from jax.experimental import pallas as pl
from jax.experimental.pallas import tpu as pltpu
from jax.experimental.pallas import tpu_sc as plsc

T = 2048
D = 768
F = 3072
E = 8
TILE = 128
NT = 24
TP = NT * TILE


def _router_meta_body(x_ref, wr_ref, logits_ref, pos_ref, inv_ref, blk_ref):
    x = x_ref[0]
    wr = wr_ref[...]
    logits = jnp.dot(x, wr, preferred_element_type=jnp.float32)
    logits_ref[...] = logits

    m = jnp.max(logits, axis=1, keepdims=True)
    lane = lax.broadcasted_iota(jnp.int32, (T, E), 1)
    ei = jnp.min(jnp.where(logits == m, lane, E), axis=1, keepdims=True)

    oh = (lane == ei).astype(jnp.float32)

    s = oh
    k = 1
    while k < T:
        s = s + jnp.concatenate(
            [jnp.zeros((k, E), jnp.float32), s[:T - k]], axis=0)
        k *= 2
    excl = s - oh
    rank = jnp.sum(excl * oh, axis=1, keepdims=True)

    counts = jnp.sum(oh, axis=0, keepdims=True)
    pc = jnp.bitwise_and(counts.astype(jnp.int32) + (TILE - 1),
                         ~(TILE - 1))
    r8 = lax.broadcasted_iota(jnp.int32, (E, E), 0)
    c8 = lax.broadcasted_iota(jnp.int32, (E, E), 1)
    upper = (r8 < c8).astype(jnp.float32)
    poff = jnp.dot(pc.astype(jnp.float32), upper,
                   preferred_element_type=jnp.float32,
                   precision=lax.Precision.HIGHEST)

    poff_t = jnp.sum(oh * poff, axis=1, keepdims=True)
    pos = (poff_t + rank).astype(jnp.int32)
    pos_ref[...] = pos

    poff_tile = poff.astype(jnp.int32) // TILE
    blk_ref[...] = jnp.zeros((1, 16), jnp.int32)
    blk_ref[:, 0:E] = poff_tile
    blk_ref[:, E:E + 1] = poff_tile[:, E - 1:E] + (pc[:, E - 1:E] // TILE)

    posf = pos.astype(jnp.float32)
    ids_i = lax.broadcasted_iota(jnp.int32, (1, T), 1)
    lhs = jnp.concatenate([
        (ids_i // 64).astype(jnp.float32),
        jnp.bitwise_and(ids_i, 63).astype(jnp.float32),
        jnp.ones((1, T), jnp.float32),
    ], axis=0)
    for c in range(NT):
        slots_i = (lax.broadcasted_iota(jnp.int32, (1, TILE), 1) + c * TILE)
        slots = slots_i.astype(jnp.float32)
        mk = (posf == slots).astype(jnp.float32)
        r = jnp.dot(lhs, mk, preferred_element_type=jnp.float32)
        invc = (r[0:1] * 64.0 + r[1:2]).astype(jnp.int32)
        valid = r[2:3] > 0.0
        fill = jnp.bitwise_and(slots_i, T - 1)
        inv_ref[:, c * TILE:(c + 1) * TILE] = jnp.where(valid, invc, fill)


_router_meta = pl.pallas_call(
    _router_meta_body,
    out_shape=[
        jax.ShapeDtypeStruct((T, E), jnp.float32),
        jax.ShapeDtypeStruct((T, 1), jnp.int32),
        jax.ShapeDtypeStruct((1, TP), jnp.int32),
        jax.ShapeDtypeStruct((1, 16), jnp.int32),
    ],
)


def _router_out_body(logits_ref, ei_ref, pw_ref):
    logits = logits_ref[...]
    m = jnp.max(logits, axis=1, keepdims=True)
    pw = 1.0 / jnp.sum(jnp.exp(logits - m), axis=1, keepdims=True)
    pw_ref[...] = jnp.broadcast_to(pw, (T, 16))
    lane = lax.broadcasted_iota(jnp.int32, (T, E), 1)
    ei_ref[...] = jnp.min(jnp.where(logits == m, lane, E), axis=1,
                          keepdims=True)


_router_out = pl.pallas_call(
    _router_out_body,
    out_shape=[
        jax.ShapeDtypeStruct((T, 1), jnp.int32),
        jax.ShapeDtypeStruct((T, 16), jnp.float32),
    ],
)


_NC, _NS = 2, 16
_NW = _NC * _NS
_RB = TP // _NW
_RC = T // _NW

def _dispatch_body(x_hbm, inv_hbm, xs_hbm, idx_v, rows_v, sem):
    wid = lax.axis_index("s") * _NC + lax.axis_index("c")
    base = wid * _RB
    pltpu.sync_copy(inv_hbm.at[pl.ds(base, _RB)], idx_v)
    pltpu.async_copy(x_hbm.at[0].at[idx_v], rows_v, sem).wait()
    pltpu.sync_copy(rows_v, xs_hbm.at[pl.ds(base, _RB)])


def _combine_body(ys_hbm, pos_hbm, pw_hbm, out_hbm, idx_v, rows_v, pw_v, sem):
    wid = lax.axis_index("s") * _NC + lax.axis_index("c")
    base = wid * _RC
    pltpu.sync_copy(pos_hbm.at[pl.ds(base, _RC)], idx_v)
    pltpu.sync_copy(pw_hbm.at[pl.ds(base, _RC)], pw_v)
    pltpu.async_copy(ys_hbm.at[idx_v], rows_v, sem).wait()

    def scale_row(r, carry):
        p = pw_v[r, 0:16]
        for j in range(D // 16):
            rows_v[r, 16 * j:16 * (j + 1)] = rows_v[r, 16 * j:16 * (j + 1)] * p
        return carry

    lax.fori_loop(0, _RC, scale_row, 0)
    pltpu.sync_copy(rows_v, out_hbm.at[pl.ds(base, _RC)])


@functools.cache
def _sc_kernels():
    mesh = plsc.VectorSubcoreMesh(core_axis_name="c", subcore_axis_name="s")
    dispatch = pl.kernel(
        _dispatch_body, mesh=mesh,
        out_type=jax.ShapeDtypeStruct((TP, D), jnp.float32),
        scratch_types=[pltpu.VMEM((_RB,), jnp.int32),
                       pltpu.VMEM((_RB, D), jnp.float32),
                       pltpu.SemaphoreType.DMA])
    combine = pl.kernel(
        _combine_body, mesh=mesh,
        out_type=jax.ShapeDtypeStruct((T, D), jnp.float32),
        scratch_types=[pltpu.VMEM((_RC,), jnp.int32),
                       pltpu.VMEM((_RC, D), jnp.float32),
                       pltpu.VMEM((_RC, 16), jnp.float32),
                       pltpu.SemaphoreType.DMA])
    return dispatch, combine


NF = 2
FB = F // NF


def _ffn_body(to_ref, xs_ref, w1_ref, b1_ref, w2_ref, b2_ref, ys_ref):
    e = pl.program_id(0)
    fi = pl.program_id(1)
    start = to_ref[e]
    end = to_ref[e + 1]

    @pl.when(fi == 0)
    def _():
        def tile0(t, carry):
            xt = xs_ref[pl.ds(t * TILE, TILE), :]
            h = jnp.dot(xt, w1_ref[0], preferred_element_type=jnp.float32)
            h = jnp.maximum(h + b1_ref[0], 0.0)
            y = jnp.dot(h, w2_ref[0], preferred_element_type=jnp.float32)
            ys_ref[pl.ds(t * TILE, TILE), :] = y
            return carry

        lax.fori_loop(start, end, tile0, 0)

    @pl.when(fi == NF - 1)
    def _():
        def tile1(t, carry):
            xt = xs_ref[pl.ds(t * TILE, TILE), :]
            h = jnp.dot(xt, w1_ref[0], preferred_element_type=jnp.float32)
            h = jnp.maximum(h + b1_ref[0], 0.0)
            y = jnp.dot(h, w2_ref[0], preferred_element_type=jnp.float32)
            ys_ref[pl.ds(t * TILE, TILE), :] = (
                ys_ref[pl.ds(t * TILE, TILE), :] + y + b2_ref[0])
            return carry

        lax.fori_loop(start, end, tile1, 0)


_ffn = pl.pallas_call(
    _ffn_body,
    grid_spec=pltpu.PrefetchScalarGridSpec(
        num_scalar_prefetch=1,
        grid=(E, NF),
        in_specs=[
            pl.BlockSpec((TP, D), lambda e, fi, to: (0, 0)),
            pl.BlockSpec((1, D, FB), lambda e, fi, to: (e, 0, fi)),
            pl.BlockSpec((1, 1, FB), lambda e, fi, to: (e, 0, fi)),
            pl.BlockSpec((1, FB, D), lambda e, fi, to: (e, fi, 0)),
            pl.BlockSpec((1, 1, D), lambda e, fi, to: (e, 0, 0)),
        ],
        out_specs=pl.BlockSpec((TP, D), lambda e, fi, to: (0, 0)),
    ),
    out_shape=jax.ShapeDtypeStruct((TP, D), jnp.float32),
    compiler_params=pltpu.CompilerParams(
        dimension_semantics=("arbitrary", "arbitrary")),
)


def kernel(hidden_states, W_router, W1, b1, W2, b2):
    shape = hidden_states.shape
    x = hidden_states.reshape(T, D)
    logits, pos, inv, blk = _router_meta(hidden_states, W_router)
    dispatch, combine = _sc_kernels()
    xs = dispatch(hidden_states, inv.reshape(TP))
    ei, pwtab = _router_out(logits)
    ys = _ffn(blk.reshape(16), xs, W1, b1.reshape(E, 1, F), W2,
              b2.reshape(E, 1, D))
    out = combine(ys, pos.reshape(T), pwtab)
    return out.reshape(shape), logits, ei.reshape(T)


# speedup vs baseline: 1.0112x; 1.0112x over previous
import functools

import jax
import jax.numpy as jnp
from jax import lax
from jax.experimental import pallas as pl
from jax.experimental.pallas import tpu as pltpu
from jax.experimental.pallas import tpu_sc as plsc

T = 2048
D = 768
F = 3072
E = 8
TILE = 128
NT = 24
TP = NT * TILE


def _router_meta_body(x_ref, wr_ref, logits_ref, ei_ref, pw_ref, pos_ref,
                      inv_ref, blk_ref):
    x = x_ref[0]
    wr = wr_ref[...]
    logits = jnp.dot(x, wr, preferred_element_type=jnp.float32)
    logits_ref[...] = logits

    m = jnp.max(logits, axis=1, keepdims=True)
    pw = 1.0 / jnp.sum(jnp.exp(logits - m), axis=1, keepdims=True)
    pw_ref[...] = jnp.broadcast_to(pw, (T, 16))

    lane = lax.broadcasted_iota(jnp.int32, (T, E), 1)
    ei = jnp.min(jnp.where(logits == m, lane, E), axis=1, keepdims=True)
    ei_ref[...] = ei

    oh = (lane == ei).astype(jnp.float32)

    s = oh
    k = 1
    while k < T:
        s = s + jnp.concatenate(
            [jnp.zeros((k, E), jnp.float32), s[:T - k]], axis=0)
        k *= 2
    excl = s - oh
    rank = jnp.sum(excl * oh, axis=1, keepdims=True)

    counts = jnp.sum(oh, axis=0, keepdims=True)
    pc = jnp.bitwise_and(counts.astype(jnp.int32) + (TILE - 1),
                         ~(TILE - 1))
    r8 = lax.broadcasted_iota(jnp.int32, (E, E), 0)
    c8 = lax.broadcasted_iota(jnp.int32, (E, E), 1)
    upper = (r8 < c8).astype(jnp.float32)
    poff = jnp.dot(pc.astype(jnp.float32), upper,
                   preferred_element_type=jnp.float32,
                   precision=lax.Precision.HIGHEST)

    poff_t = jnp.sum(oh * poff, axis=1, keepdims=True)
    pos = (poff_t + rank).astype(jnp.int32)
    pos_ref[...] = pos

    poff_tile = poff.astype(jnp.int32) // TILE
    blk_ref[...] = jnp.zeros((1, 16), jnp.int32)
    blk_ref[:, 0:E] = poff_tile
    blk_ref[:, E:E + 1] = poff_tile[:, E - 1:E] + (pc[:, E - 1:E] // TILE)

    posf = pos.astype(jnp.float32)
    ids_i = lax.broadcasted_iota(jnp.int32, (1, T), 1)
    lhs = jnp.concatenate([
        (ids_i // 64).astype(jnp.float32),
        jnp.bitwise_and(ids_i, 63).astype(jnp.float32),
        jnp.ones((1, T), jnp.float32),
    ], axis=0)
    for c in range(NT):
        slots_i = (lax.broadcasted_iota(jnp.int32, (1, TILE), 1) + c * TILE)
        slots = slots_i.astype(jnp.float32)
        mk = (posf == slots).astype(jnp.float32)
        r = jnp.dot(lhs, mk, preferred_element_type=jnp.float32)
        invc = (r[0:1] * 64.0 + r[1:2]).astype(jnp.int32)
        valid = r[2:3] > 0.0
        fill = jnp.bitwise_and(slots_i, T - 1)
        inv_ref[:, c * TILE:(c + 1) * TILE] = jnp.where(valid, invc, fill)


_router_meta = pl.pallas_call(
    _router_meta_body,
    out_shape=[
        jax.ShapeDtypeStruct((T, E), jnp.float32),
        jax.ShapeDtypeStruct((T, 1), jnp.int32),
        jax.ShapeDtypeStruct((T, 16), jnp.float32),
        jax.ShapeDtypeStruct((T, 1), jnp.int32),
        jax.ShapeDtypeStruct((1, TP), jnp.int32),
        jax.ShapeDtypeStruct((1, 16), jnp.int32),
    ],
)


_NC, _NS = 2, 16
_NW = _NC * _NS
_RB = TP // _NW
_RC = T // _NW

def _dispatch_body(x_hbm, inv_hbm, xs_hbm, idx_v, rows_v, sem):
    wid = lax.axis_index("s") * _NC + lax.axis_index("c")
    base = wid * _RB
    pltpu.sync_copy(inv_hbm.at[pl.ds(base, _RB)], idx_v)
    pltpu.async_copy(x_hbm.at[0].at[idx_v], rows_v, sem).wait()
    pltpu.sync_copy(rows_v, xs_hbm.at[pl.ds(base, _RB)])


def _combine_body(ys_hbm, pos_hbm, pw_hbm, out_hbm, idx_v, rows_v, pw_v, sem):
    wid = lax.axis_index("s") * _NC + lax.axis_index("c")
    base = wid * _RC
    pltpu.sync_copy(pos_hbm.at[pl.ds(base, _RC)], idx_v)
    pltpu.sync_copy(pw_hbm.at[pl.ds(base, _RC)], pw_v)
    pltpu.async_copy(ys_hbm.at[idx_v], rows_v, sem).wait()

    def scale_row(r, carry):
        p = pw_v[r, 0:16]
        for j in range(D // 16):
            rows_v[r, 16 * j:16 * (j + 1)] = rows_v[r, 16 * j:16 * (j + 1)] * p
        return carry

    lax.fori_loop(0, _RC, scale_row, 0)
    pltpu.sync_copy(rows_v, out_hbm.at[pl.ds(base, _RC)])


@functools.cache
def _sc_kernels():
    mesh = plsc.VectorSubcoreMesh(core_axis_name="c", subcore_axis_name="s")
    dispatch = pl.kernel(
        _dispatch_body, mesh=mesh,
        out_type=jax.ShapeDtypeStruct((TP, D), jnp.float32),
        scratch_types=[pltpu.VMEM((_RB,), jnp.int32),
                       pltpu.VMEM((_RB, D), jnp.float32),
                       pltpu.SemaphoreType.DMA])
    combine = pl.kernel(
        _combine_body, mesh=mesh,
        out_type=jax.ShapeDtypeStruct((T, D), jnp.float32),
        scratch_types=[pltpu.VMEM((_RC,), jnp.int32),
                       pltpu.VMEM((_RC, D), jnp.float32),
                       pltpu.VMEM((_RC, 16), jnp.float32),
                       pltpu.SemaphoreType.DMA])
    return dispatch, combine


NF = 2
FB = F // NF


def _ffn_body(to_ref, xs_ref, w1_ref, b1_ref, w2_ref, b2_ref, ys_ref):
    e = pl.program_id(0)
    fi = pl.program_id(1)
    start = to_ref[e]
    end = to_ref[e + 1]

    @pl.when(fi == 0)
    def _():
        def tile0(t, carry):
            xt = xs_ref[pl.ds(t * TILE, TILE), :]
            h = jnp.dot(xt, w1_ref[0], preferred_element_type=jnp.float32)
            h = jnp.maximum(h + b1_ref[0], 0.0)
            y = jnp.dot(h, w2_ref[0], preferred_element_type=jnp.float32)
            ys_ref[pl.ds(t * TILE, TILE), :] = y
            return carry

        lax.fori_loop(start, end, tile0, 0)

    @pl.when(fi == NF - 1)
    def _():
        def tile1(t, carry):
            xt = xs_ref[pl.ds(t * TILE, TILE), :]
            h = jnp.dot(xt, w1_ref[0], preferred_element_type=jnp.float32)
            h = jnp.maximum(h + b1_ref[0], 0.0)
            y = jnp.dot(h, w2_ref[0], preferred_element_type=jnp.float32)
            ys_ref[pl.ds(t * TILE, TILE), :] = (
                ys_ref[pl.ds(t * TILE, TILE), :] + y + b2_ref[0])
            return carry

        lax.fori_loop(start, end, tile1, 0)


_ffn = pl.pallas_call(
    _ffn_body,
    grid_spec=pltpu.PrefetchScalarGridSpec(
        num_scalar_prefetch=1,
        grid=(E, NF),
        in_specs=[
            pl.BlockSpec((TP, D), lambda e, fi, to: (0, 0)),
            pl.BlockSpec((1, D, FB), lambda e, fi, to: (e, 0, fi)),
            pl.BlockSpec((1, 1, FB), lambda e, fi, to: (e, 0, fi)),
            pl.BlockSpec((1, FB, D), lambda e, fi, to: (e, fi, 0)),
            pl.BlockSpec((1, 1, D), lambda e, fi, to: (e, 0, 0)),
        ],
        out_specs=pl.BlockSpec((TP, D), lambda e, fi, to: (0, 0)),
    ),
    out_shape=jax.ShapeDtypeStruct((TP, D), jnp.float32),
    compiler_params=pltpu.CompilerParams(
        dimension_semantics=("arbitrary", "arbitrary")),
)


def kernel(hidden_states, W_router, W1, b1, W2, b2):
    shape = hidden_states.shape
    logits, ei, pwtab, pos, inv, blk = _router_meta(hidden_states, W_router)
    dispatch, combine = _sc_kernels()
    xs = dispatch(hidden_states, inv.reshape(TP))
    ys = _ffn(blk.reshape(16), xs, W1, b1.reshape(E, 1, F), W2,
              b2.reshape(E, 1, D))
    out = combine(ys, pos.reshape(T), pwtab)
    return out.reshape(shape), logits, ei.reshape(T)
